# Initial kernel scaffold; baseline (speedup 1.0000x reference)
import jax
import jax.numpy as jnp
from jax.experimental import pallas as pl


def kernel(features, targets, pmarks, center):
    raise NotImplementedError("write your pallas kernel here")


# speedup vs baseline: 2.7514x; 2.7514x over previous
import functools

import jax
import jax.numpy as jnp
from jax import lax
from jax.experimental import pallas as pl
from jax.experimental.pallas import tpu as pltpu
from jax.experimental.pallas import tpu_sc as plsc

MOMENTUM = 0.99
NUM_CLASSES = 1000
B, D = 16384, 128
CP = 1024
NC, NS = 2, 16
NW = NC * NS
RPW = B // NW
RCH = 128
NCH = RPW // RCH
TROWS = CP // NS

_mesh = plsc.VectorSubcoreMesh(
    core_axis_name="c", subcore_axis_name="s", num_cores=NC, num_subcores=NS)


@functools.partial(
    pl.kernel,
    out_type=(
        jax.ShapeDtypeStruct((NC, CP, D), jnp.float32),
        jax.ShapeDtypeStruct((NC, CP, 16), jnp.float32),
    ),
    mesh=_mesh,
    scratch_types=dict(
        fbuf=pltpu.VMEM((RCH, D), jnp.float32),
        tbuf=pltpu.VMEM((RPW,), jnp.int32),
        pbuf=pltpu.VMEM((RPW,), jnp.int32),
        ibuf=pltpu.VMEM((NCH, RCH), jnp.int32),
        cbuf=pltpu.VMEM((RCH, 16), jnp.float32),
        ssum=pltpu.VMEM_SHARED((CP, D), jnp.float32),
        scnt=pltpu.VMEM_SHARED((CP, 16), jnp.float32),
    ),
)
def _scatter_kernel(features, targets, pmarks, count_src, zsum, zcnt,
                    psum, pcnt, fbuf, tbuf, pbuf, ibuf, cbuf, ssum, scnt):
    c = lax.axis_index("c")
    s = lax.axis_index("s")
    wid = c * NS + s
    base = wid * RPW

    pltpu.sync_copy(targets.at[pl.ds(base, RPW)], tbuf)
    pltpu.sync_copy(pmarks.at[pl.ds(base, RPW)], pbuf)
    pltpu.sync_copy(count_src, cbuf)

    trows = pl.ds(s * TROWS, TROWS)
    pltpu.sync_copy(zsum.at[trows], ssum.at[trows])
    pltpu.sync_copy(zcnt.at[trows], scnt.at[trows])

    for k in range(RPW // 16):
        t = tbuf[pl.ds(k * 16, 16)]
        p = pbuf[pl.ds(k * 16, 16)]
        seg = jnp.where(p == 0, t, NUM_CLASSES)
        ibuf[k // (RCH // 16), pl.ds((k % (RCH // 16)) * 16, 16)] = seg

    plsc.subcore_barrier()

    for j in range(NCH):
        pltpu.sync_copy(features.at[pl.ds(base + j * RCH, RCH)], fbuf)
        pltpu.sync_copy(fbuf, ssum.at[ibuf.at[j]], add=True)
        pltpu.sync_copy(cbuf, scnt.at[ibuf.at[j]], add=True)

    plsc.subcore_barrier()

    pltpu.sync_copy(ssum.at[trows], psum.at[c, trows])
    pltpu.sync_copy(scnt.at[trows], pcnt.at[c, trows])


@functools.partial(
    pl.kernel,
    out_type=(
        jax.ShapeDtypeStruct((NW, 16), jnp.float32),
        jax.ShapeDtypeStruct((NW, 16), jnp.float32),
    ),
    mesh=_mesh,
    scratch_types=dict(
        s0buf=pltpu.VMEM((TROWS, D), jnp.float32),
        s1buf=pltpu.VMEM((TROWS, D), jnp.float32),
        c0buf=pltpu.VMEM((TROWS, 16), jnp.float32),
        c1buf=pltpu.VMEM((TROWS, 16), jnp.float32),
        cenbuf=pltpu.VMEM((TROWS, D), jnp.float32),
        newbuf=pltpu.VMEM((TROWS, D), jnp.float32),
        stab=pltpu.VMEM_SHARED((CP, D), jnp.float32),
        tbuf=pltpu.VMEM((NCH, RCH), jnp.int32),
        pbuf=pltpu.VMEM((RPW,), jnp.int32),
        fbuf=pltpu.VMEM((RCH, D), jnp.float32),
        gbuf=pltpu.VMEM((RCH, D), jnp.float32),
        obuf=pltpu.VMEM((16,), jnp.float32),
        obuf2=pltpu.VMEM((16,), jnp.float32),
    ),
)
def _loss_kernel(psum, pcnt, center, features, targets, pmarks, out_sq, out_np,
                 s0buf, s1buf, c0buf, c1buf, cenbuf, newbuf, stab,
                 tbuf, pbuf, fbuf, gbuf, obuf, obuf2):
    c = lax.axis_index("c")
    s = lax.axis_index("s")
    wid = c * NS + s
    base = wid * RPW

    trows = pl.ds(s * TROWS, TROWS)
    pltpu.sync_copy(psum.at[0, trows], s0buf)
    pltpu.sync_copy(psum.at[1, trows], s1buf)
    pltpu.sync_copy(pcnt.at[0, trows], c0buf)
    pltpu.sync_copy(pcnt.at[1, trows], c1buf)
    pltpu.sync_copy(center.at[trows], cenbuf)

    def update_row(r, _):
        n = c0buf[r, pl.ds(0, 16)][0] + c1buf[r, pl.ds(0, 16)][0]
        has = n > 0.0
        nb = jnp.full((16,), n, jnp.float32)
        scale = (1.0 - MOMENTUM) / jnp.maximum(nb, 1.0)
        for q in range(D // 16):
            cols = pl.ds(q * 16, 16)
            sm = s0buf[r, cols] + s1buf[r, cols]
            cen = cenbuf[r, cols]
            newbuf[r, cols] = jnp.where(has, MOMENTUM * cen + scale * sm, cen)
        return 0

    lax.fori_loop(0, TROWS, update_row, 0)
    pltpu.sync_copy(newbuf, stab.at[trows])

    for j in range(NCH):
        pltpu.sync_copy(targets.at[pl.ds(base + j * RCH, RCH)], tbuf.at[j])
    pltpu.sync_copy(pmarks.at[pl.ds(base, RPW)], pbuf)

    plsc.subcore_barrier()

    acc = jnp.zeros((16,), jnp.float32)
    npv = jnp.zeros((16,), jnp.float32)
    for j in range(NCH):
        pltpu.sync_copy(features.at[pl.ds(base + j * RCH, RCH)], fbuf)
        pltpu.sync_copy(stab.at[tbuf.at[j]], gbuf)

        def grp_body(g, carry):
            a, nv = carry
            mv = jnp.where(pbuf[pl.ds(j * RCH + g * 16, 16)] != 0, 1.0, 0.0)
            nv = nv + mv
            for lane in range(16):
                m = mv[lane]
                for q in range(D // 16):
                    cols = pl.ds(q * 16, 16)
                    d = (fbuf[g * 16 + lane, cols]
                         - gbuf[g * 16 + lane, cols])
                    a = a + (d * d) * m
            return a, nv

        acc, npv = lax.fori_loop(0, RCH // 16, grp_body, (acc, npv))

    obuf[...] = acc
    pltpu.sync_copy(obuf, out_sq.at[wid])
    obuf2[...] = npv
    pltpu.sync_copy(obuf2, out_np.at[wid])


def kernel(features, targets, pmarks, center):
    count_src = jnp.zeros((RCH, 16), jnp.float32).at[:, 0].set(1.0)
    zsum = jnp.zeros((CP, D), jnp.float32)
    zcnt = jnp.zeros((CP, 16), jnp.float32)
    center_pad = jnp.zeros((CP, D), jnp.float32).at[:NUM_CLASSES].set(center)

    psum, pcnt = _scatter_kernel(features, targets, pmarks, count_src,
                                 zsum, zcnt)
    out_sq, out_np = _loss_kernel(psum, pcnt, center_pad, features, targets,
                                  pmarks)

    tot = jnp.sum(out_sq)
    n_p = jnp.sum(out_np)
    return tot / jnp.maximum(n_p * D, 1.0)
